# baseline (speedup 1.0000x reference)
import functools

import jax
import jax.numpy as jnp
from jax import lax
from jax.experimental import pallas as pl
from jax.experimental.pallas import tpu as pltpu
from jax.experimental.pallas import tpu_sc as plsc

N_ROWS = 1000000
EMB = 16
BATCH = 16384
LANES = 128

NUM_CORES = 2
NUM_SUBCORES = 16
NUM_WORKERS = NUM_CORES * NUM_SUBCORES
BPW = BATCH // NUM_WORKERS
GROUP = 16
NGROUPS = BPW // GROUP
WAVE = 8
IDX_PAD = GROUP


def _bpr_body(uidx_hbm, pidx_hbm, nidx_hbm, ut_hbm, it_hbm,
              rui_hbm, ruj_hbm, loss_hbm,
              uidx_v, pidx_v, nidx_v,
              u_s0, p_s0, n_s0, u_s1, p_s1, n_s1,
              cu, cp_, cn,
              rui_v, ruj_v, loss_v, sem0, sem1):
    wid = lax.axis_index("s") * NUM_CORES + lax.axis_index("c")
    base = wid * BPW

    pltpu.sync_copy(uidx_hbm.at[pl.ds(base, BPW)], uidx_v.at[pl.ds(0, BPW)])
    pltpu.sync_copy(pidx_hbm.at[pl.ds(base, BPW)], pidx_v.at[pl.ds(0, BPW)])
    pltpu.sync_copy(nidx_hbm.at[pl.ds(base, BPW)], nidx_v.at[pl.ds(0, BPW)])

    slots = ((u_s0, p_s0, n_s0, sem0), (u_s1, p_s1, n_s1, sem1))
    compact = (cu, cp_, cn)
    idx_refs = (uidx_v, pidx_v, nidx_v)

    lane8 = lax.iota(jnp.int32, GROUP) & (WAVE - 1)
    first8 = lax.iota(jnp.int32, GROUP) < WAVE
    rowbase = lane8 * EMB

    def wave_ivs(k, h):
        off = pl.ds(k * GROUP + h * WAVE, GROUP)
        return tuple(r[off] for r in idx_refs)

    def fire(k, h, slot):
        u_s, p_s, n_s, sem = slots[slot]
        ivs = wave_ivs(k, h)
        stages = (u_s, p_s, n_s)
        tabs = (ut_hbm, it_hbm, it_hbm)
        handles = []
        for r in range(3):
            blk = ivs[r] & ~(LANES - 1)
            for i in range(WAVE):
                b = pl.multiple_of(blk[i], LANES)
                handles.append(pltpu.async_copy(
                    tabs[r].at[:, pl.ds(b, LANES)],
                    stages[r].at[pl.ds(i * EMB, EMB)], sem))
        return handles

    def drain(slot):
        u_s, p_s, n_s, sem = slots[slot]
        dummy = ut_hbm.at[:, pl.ds(0, LANES)]
        for s in (u_s, p_s, n_s):
            for i in range(WAVE):
                pltpu.make_async_copy(
                    dummy, s.at[pl.ds(i * EMB, EMB)], sem).wait()

    def extract(k, h, slot):
        u_s, p_s, n_s, _ = slots[slot]
        ivs = wave_ivs(k, h)
        for r, stage in enumerate((u_s, p_s, n_s)):
            lanes = ivs[r] & (LANES - 1)
            dst = compact[r]
            for e in range(EMB):
                vals = plsc.load_gather(stage, [rowbase + e, lanes], mask=first8)
                plsc.store_compressed(
                    dst.at[e, pl.ds(h * WAVE, GROUP)], vals, mask=first8)

    def compute(k, loss_acc):
        acc_ui = jnp.zeros((GROUP,), jnp.float32)
        acc_uj = jnp.zeros((GROUP,), jnp.float32)
        gsl = pl.ds(0, GROUP)
        for e in range(EMB):
            u = cu[e, gsl]
            p = cp_[e, gsl]
            n = cn[e, gsl]
            acc_ui = acc_ui + u * p
            acc_uj = acc_uj + u * n
            loss_acc = loss_acc + (u * u + p * p + n * n)
        goff = pl.ds(k * GROUP, GROUP)
        rui_v[goff] = acc_ui
        ruj_v[goff] = acc_uj
        return loss_acc

    def group(k, loss_acc):
        h1 = fire(k, 1, 1)
        drain(0)
        extract(k, 0, 0)

        @pl.when(k < NGROUPS - 1)
        def _():
            fire(k + 1, 0, 0)

        for cp in h1:
            cp.wait()
        extract(k, 1, 1)
        return compute(k, loss_acc)

    fire(0, 0, 0)
    loss_acc = lax.fori_loop(0, NGROUPS, group, jnp.zeros((GROUP,), jnp.float32))
    loss_v[...] = loss_acc

    pltpu.sync_copy(rui_v, rui_hbm.at[pl.ds(base, BPW)])
    pltpu.sync_copy(ruj_v, ruj_hbm.at[pl.ds(base, BPW)])
    pltpu.sync_copy(loss_v, loss_hbm.at[wid])


@jax.jit
def _bpr_sc(uidx, pidx, nidx, ut, it):
    mesh = plsc.VectorSubcoreMesh(core_axis_name="c", subcore_axis_name="s")
    kern = functools.partial(
        pl.kernel,
        mesh=mesh,
        compiler_params=pltpu.CompilerParams(needs_layout_passes=False),
        out_type=[
            jax.ShapeDtypeStruct((BATCH,), jnp.float32),
            jax.ShapeDtypeStruct((BATCH,), jnp.float32),
            jax.ShapeDtypeStruct((NUM_WORKERS, EMB), jnp.float32),
        ],
        scratch_types=[
            pltpu.VMEM((BPW + IDX_PAD,), jnp.int32),
            pltpu.VMEM((BPW + IDX_PAD,), jnp.int32),
            pltpu.VMEM((BPW + IDX_PAD,), jnp.int32),
            pltpu.VMEM((WAVE * EMB, LANES), jnp.float32),
            pltpu.VMEM((WAVE * EMB, LANES), jnp.float32),
            pltpu.VMEM((WAVE * EMB, LANES), jnp.float32),
            pltpu.VMEM((WAVE * EMB, LANES), jnp.float32),
            pltpu.VMEM((WAVE * EMB, LANES), jnp.float32),
            pltpu.VMEM((WAVE * EMB, LANES), jnp.float32),
            pltpu.VMEM((EMB, 2 * GROUP), jnp.float32),
            pltpu.VMEM((EMB, 2 * GROUP), jnp.float32),
            pltpu.VMEM((EMB, 2 * GROUP), jnp.float32),
            pltpu.VMEM((BPW,), jnp.float32),
            pltpu.VMEM((BPW,), jnp.float32),
            pltpu.VMEM((EMB,), jnp.float32),
            pltpu.SemaphoreType.DMA,
            pltpu.SemaphoreType.DMA,
        ],
    )(_bpr_body)
    return kern(uidx, pidx, nidx, ut, it)


def kernel(users, pos_items, neg_items, user_emb, item_emb):
    users = users.astype(jnp.int32)
    pos_items = pos_items.astype(jnp.int32)
    neg_items = neg_items.astype(jnp.int32)
    ut = user_emb.T
    it = item_emb.T
    rui, ruj, loss_parts = _bpr_sc(users, pos_items, neg_items, ut, it)
    return (rui.reshape(BATCH, 1), ruj.reshape(BATCH, 1),
            jnp.sum(loss_parts))


# speedup vs baseline: 1.0646x; 1.0646x over previous
import functools

import jax
import jax.numpy as jnp
from jax import lax
from jax.experimental import pallas as pl
from jax.experimental.pallas import tpu as pltpu
from jax.experimental.pallas import tpu_sc as plsc

N_ROWS = 1000000
EMB = 16
BATCH = 16384
LANES = 128

NUM_CORES = 2
NUM_SUBCORES = 16
NUM_WORKERS = NUM_CORES * NUM_SUBCORES
BPW = BATCH // NUM_WORKERS
GROUP = 16
NGROUPS = BPW // GROUP


def _bpr_body(uidx_hbm, pidx_hbm, nidx_hbm, ut_hbm, it_hbm,
              rui_hbm, ruj_hbm, loss_hbm,
              uidx_v, pidx_v, nidx_v,
              u_s, p_s, n_s,
              rui_v, ruj_v, loss_v, sem):
    wid = lax.axis_index("s") * NUM_CORES + lax.axis_index("c")
    base = wid * BPW

    pltpu.sync_copy(uidx_hbm.at[pl.ds(base, BPW)], uidx_v)
    pltpu.sync_copy(pidx_hbm.at[pl.ds(base, BPW)], pidx_v)
    pltpu.sync_copy(nidx_hbm.at[pl.ds(base, BPW)], nidx_v)

    def group(g, loss_acc):
        goff = pl.ds(g * GROUP, GROUP)
        iv_u = uidx_v[goff]
        iv_p = pidx_v[goff]
        iv_n = nidx_v[goff]
        blk_u = iv_u & ~(LANES - 1)
        blk_p = iv_p & ~(LANES - 1)
        blk_n = iv_n & ~(LANES - 1)
        handles = []
        for i in range(GROUP):
            dst = pl.ds(i * EMB, EMB)
            bu = pl.multiple_of(blk_u[i], LANES)
            bp = pl.multiple_of(blk_p[i], LANES)
            bn = pl.multiple_of(blk_n[i], LANES)
            handles.append(pltpu.async_copy(
                ut_hbm.at[:, pl.ds(bu, LANES)], u_s.at[dst], sem))
            handles.append(pltpu.async_copy(
                it_hbm.at[:, pl.ds(bp, LANES)], p_s.at[dst], sem))
            handles.append(pltpu.async_copy(
                it_hbm.at[:, pl.ds(bn, LANES)], n_s.at[dst], sem))
        for cp in handles:
            cp.wait()

        item_rows = lax.iota(jnp.int32, GROUP) * EMB
        lane_u = iv_u & (LANES - 1)
        lane_p = iv_p & (LANES - 1)
        lane_n = iv_n & (LANES - 1)

        acc_ui = jnp.zeros((GROUP,), jnp.float32)
        acc_uj = jnp.zeros((GROUP,), jnp.float32)
        for e in range(EMB):
            rows = item_rows + e
            u = plsc.load_gather(u_s, [rows, lane_u])
            p = plsc.load_gather(p_s, [rows, lane_p])
            n = plsc.load_gather(n_s, [rows, lane_n])
            acc_ui = acc_ui + u * p
            acc_uj = acc_uj + u * n
            loss_acc = loss_acc + (u * u + p * p + n * n)
        rui_v[goff] = acc_ui
        ruj_v[goff] = acc_uj
        return loss_acc

    loss_acc = lax.fori_loop(0, NGROUPS, group, jnp.zeros((GROUP,), jnp.float32))
    loss_v[...] = loss_acc

    pltpu.sync_copy(rui_v, rui_hbm.at[pl.ds(base, BPW)])
    pltpu.sync_copy(ruj_v, ruj_hbm.at[pl.ds(base, BPW)])
    pltpu.sync_copy(loss_v, loss_hbm.at[wid])


@jax.jit
def _bpr_sc(uidx, pidx, nidx, ut, it):
    mesh = plsc.VectorSubcoreMesh(core_axis_name="c", subcore_axis_name="s")
    kern = functools.partial(
        pl.kernel,
        mesh=mesh,
        compiler_params=pltpu.CompilerParams(needs_layout_passes=False),
        out_type=[
            jax.ShapeDtypeStruct((BATCH,), jnp.float32),
            jax.ShapeDtypeStruct((BATCH,), jnp.float32),
            jax.ShapeDtypeStruct((NUM_WORKERS, EMB), jnp.float32),
        ],
        scratch_types=[
            pltpu.VMEM((BPW,), jnp.int32),
            pltpu.VMEM((BPW,), jnp.int32),
            pltpu.VMEM((BPW,), jnp.int32),
            pltpu.VMEM((GROUP * EMB, LANES), jnp.float32),
            pltpu.VMEM((GROUP * EMB, LANES), jnp.float32),
            pltpu.VMEM((GROUP * EMB, LANES), jnp.float32),
            pltpu.VMEM((BPW,), jnp.float32),
            pltpu.VMEM((BPW,), jnp.float32),
            pltpu.VMEM((EMB,), jnp.float32),
            pltpu.SemaphoreType.DMA,
        ],
    )(_bpr_body)
    return kern(uidx, pidx, nidx, ut, it)


def kernel(users, pos_items, neg_items, user_emb, item_emb):
    users = users.astype(jnp.int32)
    pos_items = pos_items.astype(jnp.int32)
    neg_items = neg_items.astype(jnp.int32)
    ut = user_emb.T
    it = item_emb.T
    rui, ruj, loss_parts = _bpr_sc(users, pos_items, neg_items, ut, it)
    return (rui.reshape(BATCH, 1), ruj.reshape(BATCH, 1),
            jnp.sum(loss_parts))
